# Initial kernel scaffold; baseline (speedup 1.0000x reference)
import functools

import jax
import jax.numpy as jnp
from jax import lax
from jax.experimental import pallas as pl
from jax.experimental.pallas import tpu as pltpu
from jax.experimental.pallas import tpu_sc as plsc

DIM = 32


def _sc_workers():
    try:
        info = plsc.get_sparse_core_info()
        return info.num_cores, info.num_subcores
    except Exception:
        return 2, 16


@functools.partial(jax.jit, static_argnums=(2, 3))
def _gather(idx_flat, table, nc, ns):
    nw = nc * ns
    b = idx_flat.shape[0]
    assert b % nw == 0
    b_per_w = b // nw
    chunk = 1024
    assert b_per_w % chunk == 0
    n_chunks = b_per_w // chunk

    mesh = plsc.VectorSubcoreMesh(core_axis_name="c", subcore_axis_name="s")

    @functools.partial(
        pl.kernel,
        mesh=mesh,
        out_type=jax.ShapeDtypeStruct((b, DIM), jnp.float32),
        scratch_types=[
            pltpu.VMEM((chunk,), jnp.int32),
            pltpu.VMEM((chunk, DIM), jnp.float32),
            pltpu.SemaphoreType.DMA,
        ],
    )
    def k(idx_hbm, table_hbm, out_hbm, idx_v, rows_v, sem):
        wid = lax.axis_index("s") * nc + lax.axis_index("c")
        base = wid * b_per_w

        def body(i, carry):
            off = base + i * chunk
            pltpu.sync_copy(idx_hbm.at[pl.ds(off, chunk)], idx_v)
            pltpu.async_copy(table_hbm.at[idx_v], rows_v, sem).wait()
            pltpu.sync_copy(rows_v, out_hbm.at[pl.ds(off, chunk)])
            return carry

        lax.fori_loop(0, n_chunks, body, 0)

    return k(idx_flat, table)


def kernel(inputs, embedding):
    nc, ns = _sc_workers()
    idx_flat = inputs.reshape(-1).astype(jnp.int32)
    out = _gather(idx_flat, embedding, nc, ns)
    return out.reshape(inputs.shape + (DIM,))


# speedup vs baseline: 4.8098x; 4.8098x over previous
import functools

import jax
import jax.numpy as jnp
from jax import lax
from jax.experimental import pallas as pl
from jax.experimental.pallas import tpu as pltpu
from jax.experimental.pallas import tpu_sc as plsc

DIM = 32


def _sc_workers():
    try:
        info = plsc.get_sparse_core_info()
        return info.num_cores, info.num_subcores
    except Exception:
        return 2, 16


@functools.partial(jax.jit, static_argnums=(2, 3))
def _gather(idx_flat, table, nc, ns):
    nw = nc * ns
    b = idx_flat.shape[0]
    assert b % nw == 0
    b_per_w = b // nw
    chunk = 1024
    assert b_per_w % chunk == 0
    n_chunks = b_per_w // chunk

    mesh = plsc.VectorSubcoreMesh(core_axis_name="c", subcore_axis_name="s")

    @functools.partial(
        pl.kernel,
        mesh=mesh,
        out_type=jax.ShapeDtypeStruct((b, DIM), jnp.float32),
        scratch_types=[
            pltpu.VMEM((chunk,), jnp.int32),
            pltpu.VMEM((chunk, DIM), jnp.float32),
            pltpu.SemaphoreType.DMA,
        ],
        compiler_params=pltpu.CompilerParams(use_tc_tiling_on_sc=False),
    )
    def k(idx_hbm, table_hbm, out_hbm, idx_v, rows_v, sem):
        wid = lax.axis_index("s") * nc + lax.axis_index("c")
        base = wid * b_per_w

        def body(i, carry):
            off = base + i * chunk
            pltpu.sync_copy(idx_hbm.at[pl.ds(off, chunk)], idx_v)
            pltpu.async_copy(table_hbm.at[idx_v], rows_v, sem).wait()
            pltpu.sync_copy(rows_v, out_hbm.at[pl.ds(off, chunk)])
            return carry

        lax.fori_loop(0, n_chunks, body, 0)

    return k(idx_flat, table)


def kernel(inputs, embedding):
    nc, ns = _sc_workers()
    idx_flat = inputs.reshape(-1).astype(jnp.int32)
    out = _gather(idx_flat, embedding, nc, ns)
    return out.reshape(inputs.shape + (DIM,))


# speedup vs baseline: 5.0244x; 1.0446x over previous
import functools

import jax
import jax.numpy as jnp
from jax import lax
from jax.experimental import pallas as pl
from jax.experimental.pallas import tpu as pltpu
from jax.experimental.pallas import tpu_sc as plsc

DIM = 32
CHUNK = 1600
NBUF = 2


def _sc_workers():
    try:
        info = plsc.get_sparse_core_info()
        return info.num_cores, info.num_subcores
    except Exception:
        return 2, 16


@functools.partial(jax.jit, static_argnums=(2, 3))
def _gather(idx_flat, table, nc, ns):
    nw = nc * ns
    b = idx_flat.shape[0]
    assert b % nw == 0
    b_per_w = b // nw
    assert b_per_w % (CHUNK * NBUF) == 0
    n_chunks = b_per_w // CHUNK
    n_groups = n_chunks // NBUF

    mesh = plsc.VectorSubcoreMesh(core_axis_name="c", subcore_axis_name="s")

    @functools.partial(
        pl.kernel,
        mesh=mesh,
        out_type=jax.ShapeDtypeStruct((b, DIM), jnp.float32),
        scratch_types=[
            pltpu.VMEM((NBUF, CHUNK), jnp.int32),
            pltpu.VMEM((NBUF, CHUNK, DIM), jnp.float32),
        ]
        + [pltpu.SemaphoreType.DMA] * (3 * NBUF),
        compiler_params=pltpu.CompilerParams(use_tc_tiling_on_sc=False),
    )
    def k(idx_hbm, table_hbm, out_hbm, idx_v, rows_v, *sems):
        idx_sem = sems[0:NBUF]
        g_sem = sems[NBUF : 2 * NBUF]
        o_sem = sems[2 * NBUF : 3 * NBUF]
        wid = lax.axis_index("s") * nc + lax.axis_index("c")
        base = wid * b_per_w

        def idx_slice(c):
            return idx_hbm.at[pl.ds(base + c * CHUNK, CHUNK)]

        def out_slice(c):
            return out_hbm.at[pl.ds(base + c * CHUNK, CHUNK)]

        for bi in range(NBUF):
            pltpu.async_copy(idx_slice(bi), idx_v.at[bi], idx_sem[bi])

        def group(g, first):
            gathers = []
            for bi in range(NBUF):
                c = g * NBUF + bi
                pltpu.make_async_copy(idx_slice(c), idx_v.at[bi], idx_sem[bi]).wait()
                if not first:
                    pltpu.make_async_copy(rows_v.at[bi], out_slice(c), o_sem[bi]).wait()
                gathers.append(
                    pltpu.async_copy(table_hbm.at[idx_v.at[bi]], rows_v.at[bi], g_sem[bi])
                )
            for bi in range(NBUF):
                c = g * NBUF + bi
                gathers[bi].wait()
                pltpu.async_copy(rows_v.at[bi], out_slice(c), o_sem[bi])
                nxt = jnp.minimum(c + NBUF, n_chunks - 1)
                pltpu.async_copy(idx_slice(nxt), idx_v.at[bi], idx_sem[bi])

        group(0, True)

        def body(g, carry):
            group(g, False)
            return carry

        lax.fori_loop(1, n_groups, body, 0)

        for bi in range(NBUF):
            pltpu.make_async_copy(idx_slice(0), idx_v.at[bi], idx_sem[bi]).wait()
            pltpu.make_async_copy(rows_v.at[bi], out_slice(0), o_sem[bi]).wait()

    return k(idx_flat, table)


def kernel(inputs, embedding):
    nc, ns = _sc_workers()
    idx_flat = inputs.reshape(-1).astype(jnp.int32)
    out = _gather(idx_flat, embedding, nc, ns)
    return out.reshape(inputs.shape + (DIM,))


# speedup vs baseline: 5.0450x; 1.0041x over previous
import functools

import jax
import jax.numpy as jnp
from jax import lax
from jax.experimental import pallas as pl
from jax.experimental.pallas import tpu as pltpu
from jax.experimental.pallas import tpu_sc as plsc

DIM = 32
NB = 512
LANES = 16


def _sc_workers():
    try:
        info = plsc.get_sparse_core_info()
        return info.num_cores, info.num_subcores
    except Exception:
        return 2, 16


@functools.partial(jax.jit, static_argnums=(2, 3))
def _gather_t(idx_t, table, nc, ns):
    nw = nc * ns
    seq, nbatch = idx_t.shape
    assert nbatch % NB == 0
    blocks_per_s = nbatch // NB
    assert blocks_per_s == 32
    n_blocks = seq * blocks_per_s
    assert n_blocks % nw == 0
    blk_per_w = n_blocks // nw
    assert blk_per_w % 2 == 0
    ntr = DIM // 8
    ntc = NB // 128

    mesh = plsc.VectorSubcoreMesh(core_axis_name="c", subcore_axis_name="s")

    @functools.partial(
        pl.kernel,
        mesh=mesh,
        out_type=jax.ShapeDtypeStruct((seq, ntr, nbatch // 128, 8, 128), jnp.float32),
        scratch_types=[
            pltpu.VMEM((2, NB), jnp.int32),
            pltpu.VMEM((2, NB, DIM), jnp.float32),
            pltpu.VMEM((2, ntr, ntc, 8, 128), jnp.float32),
        ]
        + [pltpu.SemaphoreType.DMA] * 6,
        compiler_params=pltpu.CompilerParams(
            use_tc_tiling_on_sc=False, needs_layout_passes=False
        ),
    )
    def k(idx_hbm, table_hbm, out_hbm, idx_v, g_v, w_v, *sems):
        idx_sem = sems[0:2]
        g_sem = sems[2:4]
        o_sem = sems[4:6]
        wid = lax.axis_index("s") * nc + lax.axis_index("c")
        blk0 = wid * blk_per_w
        last = blk_per_w - 1

        tr_lo = lax.shift_right_logical(lax.iota(jnp.int32, LANES), 3)
        sl_vec = lax.iota(jnp.int32, LANES) & 7
        tr_hi = tr_lo + 2

        def coords(i):
            beta = blk0 + i
            s = lax.shift_right_logical(beta, 5)
            bq = beta & (blocks_per_s - 1)
            return s, bq

        def idx_copy(bi, i):
            s, bq = coords(i)
            return pltpu.make_async_copy(
                idx_hbm.at[s, pl.ds(bq * NB, NB)], idx_v.at[bi], idx_sem[bi]
            )

        def gather(bi, i):
            return pltpu.async_copy(
                table_hbm.at[idx_v.at[bi]], g_v.at[bi], g_sem[bi]
            )

        def out_copies(bi, i):
            s, bq = coords(i)
            return [
                pltpu.make_async_copy(
                    w_v.at[bi, tr],
                    out_hbm.at[s, tr, pl.ds(bq * ntc, ntc)],
                    o_sem[bi],
                )
                for tr in range(ntr)
            ]

        def transpose(bi):
            g2 = g_v.at[bi]
            w4 = w_v.at[bi]

            def body(j, carry):
                tcq = jnp.broadcast_to(lax.shift_right_logical(j, 7), (LANES,))
                ln = jnp.broadcast_to(j & 127, (LANES,))
                v0 = g2[j, pl.ds(0, LANES)]
                v1 = g2[j, pl.ds(LANES, LANES)]
                plsc.store_scatter(w4, [tr_lo, tcq, sl_vec, ln], v0)
                plsc.store_scatter(w4, [tr_hi, tcq, sl_vec, ln], v1)
                return carry

            lax.fori_loop(0, NB, body, 0)

        def step(blk, bi, bj, first):
            nxt = jnp.minimum(blk + 1, last)
            nxt2 = jnp.minimum(blk + 2, last)
            idx_copy(bj, nxt).wait()
            gather(bj, nxt)
            pltpu.make_async_copy(
                table_hbm.at[idx_v.at[bi]], g_v.at[bi], g_sem[bi]
            ).wait()
            idx_copy(bi, nxt2).start()
            if not first:
                for cp in out_copies(bi, blk):
                    cp.wait()
            transpose(bi)
            for cp in out_copies(bi, blk):
                cp.start()

        idx_copy(0, jnp.int32(0)).start()
        idx_copy(1, jnp.int32(1)).start()
        idx_copy(0, jnp.int32(0)).wait()
        gather(0, jnp.int32(0))
        step(jnp.int32(0), 0, 1, True)
        step(jnp.int32(1), 1, 0, True)

        def body(g, carry):
            step(2 * g, 0, 1, False)
            step(2 * g + 1, 1, 0, False)
            return carry

        lax.fori_loop(1, blk_per_w // 2, body, 0)

        pltpu.make_async_copy(
            table_hbm.at[idx_v.at[0]], g_v.at[0], g_sem[0]
        ).wait()
        idx_copy(1, jnp.int32(last)).wait()
        for bi in range(2):
            for cp in out_copies(bi, jnp.int32(last)):
                cp.wait()

    return k(idx_t, table)


def kernel(inputs, embedding):
    nc, ns = _sc_workers()
    idx_t = inputs.astype(jnp.int32).T
    nrows, seq = inputs.shape
    out = _gather_t(idx_t, embedding, nc, ns)
    return out.transpose(2, 4, 0, 1, 3).reshape(nrows, seq, DIM)


# speedup vs baseline: 15.2174x; 3.0163x over previous
import functools

import jax
import jax.numpy as jnp
from jax import lax
from jax.experimental import pallas as pl
from jax.experimental.pallas import tpu as pltpu
from jax.experimental.pallas import tpu_sc as plsc

DIM = 32
NB = 512
LANES = 16


def _sc_workers():
    try:
        info = plsc.get_sparse_core_info()
        return info.num_cores, info.num_subcores
    except Exception:
        return 2, 16


@functools.partial(jax.jit, static_argnums=(2, 3))
def _gather_t(idx_t, table, nc, ns):
    nw = nc * ns
    seq, nbatch = idx_t.shape
    assert nbatch % NB == 0
    blocks_per_s = nbatch // NB
    assert blocks_per_s == 32
    n_blocks = seq * blocks_per_s
    assert n_blocks % nw == 0
    blk_per_w = n_blocks // nw
    assert blk_per_w % 2 == 0
    ntr = DIM // 8
    ntc = NB // 128

    mesh = plsc.VectorSubcoreMesh(core_axis_name="c", subcore_axis_name="s")

    @functools.partial(
        pl.kernel,
        mesh=mesh,
        out_type=jax.ShapeDtypeStruct((seq * ntr * (nbatch // 128) * 8 * 128,), jnp.float32),
        scratch_types=[
            pltpu.VMEM((2, NB), jnp.int32),
            pltpu.VMEM((2, NB, DIM), jnp.float32),
            pltpu.VMEM((2, ntr * ntc * 8 * 128), jnp.float32),
        ]
        + [pltpu.SemaphoreType.DMA] * 6,
        compiler_params=pltpu.CompilerParams(
            use_tc_tiling_on_sc=False, needs_layout_passes=False
        ),
    )
    def k(idx_hbm, table_hbm, out_hbm, idx_v, g_v, w_v, *sems):
        idx_sem = sems[0:2]
        g_sem = sems[2:4]
        o_sem = sems[4:6]
        wid = lax.axis_index("s") * nc + lax.axis_index("c")
        blk0 = wid * blk_per_w
        last = blk_per_w - 1

        tr_lo = lax.shift_right_logical(lax.iota(jnp.int32, LANES), 3)
        sl_vec = lax.iota(jnp.int32, LANES) & 7
        tr_hi = tr_lo + 2

        def coords(i):
            beta = blk0 + i
            s = lax.shift_right_logical(beta, 5)
            bq = beta & (blocks_per_s - 1)
            return s, bq

        def idx_copy(bi, i):
            s, bq = coords(i)
            return pltpu.make_async_copy(
                idx_hbm.at[s, pl.ds(bq * NB, NB)], idx_v.at[bi], idx_sem[bi]
            )

        def gather(bi, i):
            return pltpu.async_copy(
                table_hbm.at[idx_v.at[bi]], g_v.at[bi], g_sem[bi]
            )

        p_lo = tr_lo * (ntc * 1024) + sl_vec * 128
        p_hi = p_lo + 2 * (ntc * 1024)
        tile_sz = ntc * 8 * 128

        def out_copies(bi, i):
            s, bq = coords(i)
            row = (s * ntr) * (nbatch // 128) + bq * ntc
            return [
                pltpu.make_async_copy(
                    w_v.at[bi, pl.ds(tr * tile_sz, tile_sz)],
                    out_hbm.at[pl.ds((row + tr * (nbatch // 128)) * 1024, tile_sz)],
                    o_sem[bi],
                )
                for tr in range(ntr)
            ]

        def transpose(bi):
            g2 = g_v.at[bi]
            w1 = w_v.at[bi]

            @functools.partial(plsc.parallel_loop, 0, NB, unroll=8)
            def body(j):
                t = jnp.broadcast_to(j + lax.shift_right_logical(j, 7) * 896, (LANES,))
                v0 = g2[j, pl.ds(0, LANES)]
                v1 = g2[j, pl.ds(LANES, LANES)]
                plsc.store_scatter(w1, [p_lo + t], v0)
                plsc.store_scatter(w1, [p_hi + t], v1)

        def step(blk, bi, bj, first):
            nxt = jnp.minimum(blk + 1, last)
            nxt2 = jnp.minimum(blk + 2, last)
            idx_copy(bj, nxt).wait()
            gather(bj, nxt)
            pltpu.make_async_copy(
                table_hbm.at[idx_v.at[bi]], g_v.at[bi], g_sem[bi]
            ).wait()
            idx_copy(bi, nxt2).start()
            if not first:
                for cp in out_copies(bi, blk):
                    cp.wait()
            transpose(bi)
            for cp in out_copies(bi, blk):
                cp.start()

        idx_copy(0, jnp.int32(0)).start()
        idx_copy(1, jnp.int32(1)).start()
        idx_copy(0, jnp.int32(0)).wait()
        gather(0, jnp.int32(0))
        step(jnp.int32(0), 0, 1, True)
        step(jnp.int32(1), 1, 0, True)

        def body(g, carry):
            step(2 * g, 0, 1, False)
            step(2 * g + 1, 1, 0, False)
            return carry

        lax.fori_loop(1, blk_per_w // 2, body, 0)

        pltpu.make_async_copy(
            table_hbm.at[idx_v.at[0]], g_v.at[0], g_sem[0]
        ).wait()
        idx_copy(1, jnp.int32(last)).wait()
        for bi in range(2):
            for cp in out_copies(bi, jnp.int32(last)):
                cp.wait()

    return k(idx_t, table)


def kernel(inputs, embedding):
    nc, ns = _sc_workers()
    idx_t = inputs.astype(jnp.int32).T
    nrows, seq = inputs.shape
    out = _gather_t(idx_t, embedding, nc, ns)
    out = out.reshape(seq, DIM // 8, nrows // 128, 8, 128)
    return out.transpose(2, 4, 0, 1, 3).reshape(nrows, seq, DIM)


# speedup vs baseline: 28.9144x; 1.9001x over previous
import functools

import jax
import jax.numpy as jnp
from jax import lax
from jax.experimental import pallas as pl
from jax.experimental.pallas import tpu as pltpu
from jax.experimental.pallas import tpu_sc as plsc

DIM = 32
NB = 512
LANES = 16


def _sc_workers():
    try:
        info = plsc.get_sparse_core_info()
        return info.num_cores, info.num_subcores
    except Exception:
        return 2, 16


@functools.partial(jax.jit, static_argnums=(2, 3))
def _detile_table(emb_t, tail_flat, nc, ns):
    nw = nc * ns
    d, v = emb_t.shape
    assert d == DIM
    cols = 512
    full = (v // cols) * cols
    tail = v - full
    n_blocks = 2 * (-(-(full // cols) // (2 * nw)))
    stride_w = (full // cols // nw) * cols

    mesh = plsc.VectorSubcoreMesh(core_axis_name="c", subcore_axis_name="s")

    @functools.partial(
        pl.kernel,
        mesh=mesh,
        out_type=jax.ShapeDtypeStruct((v * DIM,), jnp.float32),
        scratch_types=[
            pltpu.VMEM((2, DIM, cols), jnp.float32),
            pltpu.VMEM((2, cols * DIM), jnp.float32),
        ]
        + [pltpu.SemaphoreType.DMA] * 4,
        compiler_params=pltpu.CompilerParams(
            use_tc_tiling_on_sc=True, needs_layout_passes=False
        ),
    )
    def k(emb_hbm, tail_hbm, out_hbm, in_v, t_v, *sems):
        i_sem = sems[0:2]
        o_sem = sems[2:4]
        wid = lax.axis_index("s") * nc + lax.axis_index("c")
        c_base = wid * stride_w
        cmax = full - cols

        def c0_of(b):
            return jnp.minimum(c_base + b * cols, cmax)

        def in_copy(bi, b):
            return pltpu.make_async_copy(
                emb_hbm.at[:, pl.ds(c0_of(b), cols)], in_v.at[bi], i_sem[bi]
            )

        def out_copy(bi, b):
            return pltpu.make_async_copy(
                t_v.at[bi], out_hbm.at[pl.ds(c0_of(b) * DIM, cols * DIM)], o_sem[bi]
            )

        p_vec = lax.iota(jnp.int32, LANES) * DIM

        def transpose(bi):
            g2 = in_v.at[bi]
            w1 = t_v.at[bi]

            @functools.partial(plsc.parallel_loop, 0, DIM * (cols // LANES), unroll=8)
            def body(it):
                dd = lax.shift_right_logical(it, 5)
                g = it & 31
                vv = g2[dd, pl.ds(g * LANES, LANES)]
                t = jnp.broadcast_to(g * (LANES * DIM) + dd, (LANES,))
                plsc.store_scatter(w1, [p_vec + t], vv)

        def step(b, bi, bj, first):
            in_copy(bj, b + 1).start()
            in_copy(bi, b).wait()
            if not first:
                out_copy(bi, b).wait()
            transpose(bi)
            out_copy(bi, b).start()

        in_copy(0, jnp.int32(0)).start()
        step(jnp.int32(0), 0, 1, True)
        step(jnp.int32(1), 1, 0, True)

        def body(b, carry):
            step(2 * b, 0, 1, False)
            step(2 * b + 1, 1, 0, False)
            return carry

        lax.fori_loop(1, n_blocks // 2, body, 0)

        in_copy(0, jnp.int32(0)).wait()
        out_copy(0, jnp.int32(0)).wait()
        out_copy(1, jnp.int32(0)).wait()

        @pl.when(wid == 0)
        def _tail():
            pltpu.sync_copy(tail_hbm, out_hbm.at[pl.ds(full * DIM, tail * DIM)])

    return k(emb_t, tail_flat)


@functools.partial(jax.jit, static_argnums=(2, 3))
def _gather_t(idx_t, table, nc, ns):
    nw = nc * ns
    seq, nbatch = idx_t.shape
    assert nbatch % NB == 0
    blocks_per_s = nbatch // NB
    assert blocks_per_s == 32
    n_blocks = seq * blocks_per_s
    assert n_blocks % nw == 0
    blk_per_w = n_blocks // nw
    assert blk_per_w % 2 == 0
    ntr = DIM // 8
    ntc = NB // 128

    mesh = plsc.VectorSubcoreMesh(core_axis_name="c", subcore_axis_name="s")

    @functools.partial(
        pl.kernel,
        mesh=mesh,
        out_type=jax.ShapeDtypeStruct((seq * ntr * (nbatch // 128) * 8 * 128,), jnp.float32),
        scratch_types=[
            pltpu.VMEM((2, NB), jnp.int32),
            pltpu.VMEM((2, NB, DIM), jnp.float32),
            pltpu.VMEM((2, ntr * ntc * 8 * 128), jnp.float32),
        ]
        + [pltpu.SemaphoreType.DMA] * 6,
        compiler_params=pltpu.CompilerParams(
            use_tc_tiling_on_sc=False, needs_layout_passes=False
        ),
    )
    def k(idx_hbm, table_hbm, out_hbm, idx_v, g_v, w_v, *sems):
        idx_sem = sems[0:2]
        g_sem = sems[2:4]
        o_sem = sems[4:6]
        wid = lax.axis_index("s") * nc + lax.axis_index("c")
        blk0 = wid * blk_per_w
        last = blk_per_w - 1

        tr_lo = lax.shift_right_logical(lax.iota(jnp.int32, LANES), 3)
        sl_vec = lax.iota(jnp.int32, LANES) & 7
        tr_hi = tr_lo + 2

        def coords(i):
            beta = blk0 + i
            s = lax.shift_right_logical(beta, 5)
            bq = beta & (blocks_per_s - 1)
            return s, bq

        def idx_copy(bi, i):
            s, bq = coords(i)
            return pltpu.make_async_copy(
                idx_hbm.at[s, pl.ds(bq * NB, NB)], idx_v.at[bi], idx_sem[bi]
            )

        def gather(bi, i):
            return pltpu.async_copy(
                table_hbm.at[idx_v.at[bi]], g_v.at[bi], g_sem[bi]
            )

        p_lo = tr_lo * (ntc * 1024) + sl_vec * 128
        p_hi = p_lo + 2 * (ntc * 1024)
        tile_sz = ntc * 8 * 128

        def out_copies(bi, i):
            s, bq = coords(i)
            row = (s * ntr) * (nbatch // 128) + bq * ntc
            return [
                pltpu.make_async_copy(
                    w_v.at[bi, pl.ds(tr * tile_sz, tile_sz)],
                    out_hbm.at[pl.ds((row + tr * (nbatch // 128)) * 1024, tile_sz)],
                    o_sem[bi],
                )
                for tr in range(ntr)
            ]

        def transpose(bi):
            g2 = g_v.at[bi]
            w1 = w_v.at[bi]

            @functools.partial(plsc.parallel_loop, 0, NB, unroll=8)
            def body(j):
                t = jnp.broadcast_to(j + lax.shift_right_logical(j, 7) * 896, (LANES,))
                v0 = g2[j, pl.ds(0, LANES)]
                v1 = g2[j, pl.ds(LANES, LANES)]
                plsc.store_scatter(w1, [p_lo + t], v0)
                plsc.store_scatter(w1, [p_hi + t], v1)

        def step(blk, bi, bj, first):
            nxt = jnp.minimum(blk + 1, last)
            nxt2 = jnp.minimum(blk + 2, last)
            idx_copy(bj, nxt).wait()
            gather(bj, nxt)
            pltpu.make_async_copy(
                table_hbm.at[idx_v.at[bi]], g_v.at[bi], g_sem[bi]
            ).wait()
            idx_copy(bi, nxt2).start()
            if not first:
                for cp in out_copies(bi, blk):
                    cp.wait()
            transpose(bi)
            for cp in out_copies(bi, blk):
                cp.start()

        idx_copy(0, jnp.int32(0)).start()
        idx_copy(1, jnp.int32(1)).start()
        idx_copy(0, jnp.int32(0)).wait()
        gather(0, jnp.int32(0))
        step(jnp.int32(0), 0, 1, True)
        step(jnp.int32(1), 1, 0, True)

        def body(g, carry):
            step(2 * g, 0, 1, False)
            step(2 * g + 1, 1, 0, False)
            return carry

        lax.fori_loop(1, blk_per_w // 2, body, 0)

        pltpu.make_async_copy(
            table_hbm.at[idx_v.at[0]], g_v.at[0], g_sem[0]
        ).wait()
        idx_copy(1, jnp.int32(last)).wait()
        for bi in range(2):
            for cp in out_copies(bi, jnp.int32(last)):
                cp.wait()

    return k(idx_t, table)


def kernel(inputs, embedding):
    nc, ns = _sc_workers()
    idx_t = inputs.astype(jnp.int32).T
    nrows, seq = inputs.shape
    n_full = (embedding.shape[0] // 512) * 512
    tail_flat = embedding[n_full:].reshape(-1)
    table = _detile_table(embedding.T, tail_flat, nc, ns).reshape(-1, DIM)
    out = _gather_t(idx_t, table, nc, ns)
    out = out.reshape(seq, DIM // 8, nrows // 128, 8, 128)
    return out.transpose(2, 4, 0, 1, 3).reshape(nrows, seq, DIM)
